# baseline (speedup 1.0000x reference)
import numpy as np
import jax
import jax.numpy as jnp
from jax.experimental import pallas as pl
from jax.experimental.pallas import tpu as pltpu

K = 10
EPS = 1e-5
KM_ITERS = 5
PTILE = 1024
_NEG = -1e30
_F32 = jnp.float32


def _split(x):
    hi = x.astype(jnp.bfloat16)
    lo = (x - hi.astype(_F32)).astype(jnp.bfloat16)
    return hi, lo


def _bag_kernel(bags_ref, W_enc_ref, b_enc_ref, Wa1_ref, ba1_ref, Wa2_ref,
                ba2_ref, g_r_ref, be_r_ref, W_rh_ref, b_rh_ref, Ws1_ref,
                bs1_ref, Ws2_ref, bs2_ref, g_s_ref, be_s_ref, W_shp_ref,
                b_shp_ref, out_ref, ehi_ref, elo_ref, a_ref):
    b = pl.program_id(0)
    pt = pl.program_id(1)
    nb = pl.num_programs(0)
    npt = pl.num_programs(1)
    Z = ehi_ref.shape[1]
    P = ehi_ref.shape[0] // nb

    tile = bags_ref[0]
    tsz = tile.shape[0]
    e = jnp.maximum(
        jnp.dot(tile, W_enc_ref[...], preferred_element_type=_F32)
        + b_enc_ref[...], 0.0)
    h = jnp.tanh(
        jnp.dot(e, Wa1_ref[...], preferred_element_type=_F32)
        + ba1_ref[...])
    at = (jax.lax.dot_general(Wa2_ref[...], h, (((0,), (1,)), ((), ())),
                              preferred_element_type=_F32)
          + ba2_ref[...])
    e_hi, e_lo = _split(e)
    row0 = b * P + pt * tsz
    ehi_ref[pl.ds(row0, tsz), :] = e_hi
    elo_ref[pl.ds(row0, tsz), :] = e_lo
    a_ref[:, pl.ds(row0, tsz)] = at

    @pl.when((b == nb - 1) & (pt == npt - 1))
    def _phase2():
        iota_kp = jax.lax.broadcasted_iota(jnp.int32, (K, P), 0)
        idx = np.linspace(0, P - 1, K).astype(np.int32)

        ehi = [ehi_ref[pl.ds(bb * P, P), :] for bb in range(nb)]
        elo = [elo_ref[pl.ds(bb * P, P), :] for bb in range(nb)]

        def assign(cent, bb):
            csq = jnp.sum(cent * cent, axis=1, keepdims=True)
            c_hi, c_lo = _split(-2.0 * cent)
            dims = (((1,), (1,)), ((), ()))
            dg = lambda a, bm: jax.lax.dot_general(
                a, bm, dims, preferred_element_type=_F32)
            d = (csq + dg(c_hi, ehi[bb]) + dg(c_hi, elo[bb])
                 + dg(c_lo, ehi[bb]))
            dmin = jnp.min(d, axis=0, keepdims=True)
            amin = jnp.min(jnp.where(d == dmin, iota_kp, K),
                           axis=0, keepdims=True)
            return (iota_kp == amin).astype(_F32)

        cents = [
            jnp.concatenate(
                [ehi_ref[bb * P + i:bb * P + i + 1, :].astype(_F32)
                 + elo_ref[bb * P + i:bb * P + i + 1, :].astype(_F32)
                 for i in idx], axis=0)
            for bb in range(nb)]

        for _ in range(KM_ITERS):
            onehots = [assign(cents[bb], bb) for bb in range(nb)]
            new_cents = []
            for bb in range(nb):
                oh = onehots[bb]
                cnt = jnp.sum(oh, axis=1, keepdims=True)
                ob = oh.astype(jnp.bfloat16)
                pdims = (((1,), (0,)), ((), ()))
                s = (jax.lax.dot_general(ob, ehi[bb], pdims,
                                         preferred_element_type=_F32)
                     + jax.lax.dot_general(ob, elo[bb], pdims,
                                           preferred_element_type=_F32))
                new_cents.append(s / jnp.maximum(cnt, 1.0))
            cents = new_cents
        onehots = [assign(cents[bb], bb) for bb in range(nb)]

        for bb in range(nb):
            onehot = onehots[bb]
            a_row = a_ref[:, pl.ds(bb * P, P)]
            A = jnp.where(onehot > 0.0, a_row, _NEG)
            m = jnp.max(A, axis=1, keepdims=True)
            E = onehot * jnp.exp(A - m)
            sseg = jnp.sum(E, axis=1, keepdims=True)
            W = E / jnp.maximum(sseg, 1e-12)

            W_hi, W_lo = _split(W)
            pdims = (((1,), (0,)), ((), ()))
            dg = lambda a, bm: jax.lax.dot_general(
                a, bm, pdims, preferred_element_type=_F32)
            region = (dg(W_hi, ehi[bb]) + dg(W_hi, elo[bb])
                      + dg(W_lo, ehi[bb]))
            reg_bn = (region * (1.0 / np.sqrt(1.0 + EPS)) * g_r_ref[...]
                      + be_r_ref[...])
            reg_out = (jnp.dot(reg_bn, W_rh_ref[...],
                               preferred_element_type=_F32)
                       + b_rh_ref[...])

            hs = jnp.tanh(
                jnp.dot(reg_out, Ws1_ref[...], preferred_element_type=_F32)
                + bs1_ref[...])
            sa = (jnp.dot(hs, Ws2_ref[...], preferred_element_type=_F32)
                  + bs2_ref[...])
            aw = jnp.exp(sa - jnp.max(sa))
            aw = aw / jnp.sum(aw)
            slide = jnp.sum(aw * reg_out, axis=0, keepdims=True)
            slide_bn = (slide * (1.0 / np.sqrt(1.0 + EPS)) * g_s_ref[...]
                        + be_s_ref[...])
            out_ref[bb] = (jnp.dot(slide_bn, W_shp_ref[...],
                                   preferred_element_type=_F32)
                           + b_shp_ref[...])


def kernel(bags, W_enc, b_enc, Wa1, ba1, Wa2, ba2, g_r, be_r, W_rh, b_rh,
           Ws1, bs1, Ws2, bs2, g_s, be_s, W_sh, b_sh):
    B, P, F = bags.shape
    Z = W_enc.shape[1]
    NOUT = W_sh.shape[1]
    OPAD = 128
    ptile = min(PTILE, P)
    npt = P // ptile
    assert P % ptile == 0

    W_shp = jnp.zeros((Z, OPAD), jnp.float32).at[:, :NOUT].set(W_sh)
    b_shp = jnp.zeros((1, OPAD), jnp.float32).at[:, :NOUT].set(b_sh[None, :])

    full = lambda *shape: pl.BlockSpec(shape, lambda b, pt: tuple(0 for _ in shape))
    out = pl.pallas_call(
        _bag_kernel,
        grid=(B, npt),
        in_specs=[
            pl.BlockSpec((1, ptile, F), lambda b, pt: (b, pt, 0)),
            full(F, Z), full(1, Z),
            full(Z, Z), full(1, Z),
            full(Z, 1), full(1, 1),
            full(1, Z), full(1, Z),
            full(Z, Z), full(1, Z),
            full(Z, Z), full(1, Z),
            full(Z, 1), full(1, 1),
            full(1, Z), full(1, Z),
            full(Z, OPAD), full(1, OPAD),
        ],
        out_specs=pl.BlockSpec((B, 1, OPAD), lambda b, pt: (0, 0, 0)),
        out_shape=jax.ShapeDtypeStruct((B, 1, OPAD), jnp.float32),
        scratch_shapes=[
            pltpu.VMEM((B * P, Z), jnp.bfloat16),
            pltpu.VMEM((B * P, Z), jnp.bfloat16),
            pltpu.VMEM((1, B * P), jnp.float32),
        ],
        compiler_params=pltpu.CompilerParams(
            dimension_semantics=("arbitrary", "arbitrary"),
        ),
    )(bags, W_enc, b_enc[None, :], Wa1, ba1[None, :], Wa2, ba2[None, :],
      g_r[None, :], be_r[None, :], W_rh, b_rh[None, :], Ws1, bs1[None, :],
      Ws2, bs2[None, :], g_s[None, :], be_s[None, :], W_shp, b_shp)
    return out[:, 0, :NOUT]
